# Initial kernel scaffold; baseline (speedup 1.0000x reference)
import jax
import jax.numpy as jnp
from jax.experimental import pallas as pl


def kernel(X_A, X_B, prototype_base, Wc, bc, WA, bA, WB, bB):
    raise NotImplementedError("write your pallas kernel here")


# speedup vs baseline: 1.4259x; 1.4259x over previous
import functools
import math

import jax
import jax.numpy as jnp
from jax import lax
from jax.experimental import pallas as pl
from jax.experimental.pallas import tpu as pltpu
from jax.experimental.pallas import tpu_sc as plsc

_NW = 32
_CH = 16


def _ctx_kernel(xa_ref, xb_ref, out_ref):
    d = xa_ref.shape[2]
    out_ref[0, 0, :d] = jnp.mean(xa_ref[0], axis=0)
    out_ref[0, 0, d:] = jnp.mean(xb_ref[0], axis=0)


def _proto_kernel(ctx_ref, wc_ref, bc_ref, base_ref, out_ref):
    for i in range(wc_ref.shape[0]):
        off = jax.lax.dot_general(
            ctx_ref[...], wc_ref[i], (((1,), (1,)), ((), ())),
            preferred_element_type=jnp.float32)
        out_ref[i] = off + bc_ref[i] + base_ref[i]


def _logits_kernel(x_ref, w_ref, b_ref, p_ref, out_ref, *, scale):
    x = x_ref[0]
    xp = jax.lax.dot_general(
        x, w_ref[...], (((1,), (1,)), ((), ())),
        preferred_element_type=jnp.float32) + b_ref[...]
    out_ref[0] = jnp.dot(xp, p_ref[0], preferred_element_type=jnp.float32) * scale


def _merge32(ka, va, kb, vb):
    rk = jnp.flip(kb, 0)
    rv = jnp.flip(vb, 0)
    m = ka >= rk
    hk = jnp.where(m, ka, rk)
    hv = jnp.where(m, va, rv)
    lk = jnp.where(m, rk, ka)
    lv = jnp.where(m, rv, va)
    hk, hv = plsc.sort_key_val(hk, hv, descending=True)
    lk, lv = plsc.sort_key_val(lk, lv, descending=True)
    return hk, hv, lk, lv


def _top32(a0, av0, a1, av1, b0, bv0, b1, bv1):
    r0k = jnp.flip(b1, 0)
    r0v = jnp.flip(bv1, 0)
    r1k = jnp.flip(b0, 0)
    r1v = jnp.flip(bv0, 0)
    m0 = a0 >= r0k
    h0 = jnp.where(m0, a0, r0k)
    hv0 = jnp.where(m0, av0, r0v)
    m1 = a1 >= r1k
    h1 = jnp.where(m1, a1, r1k)
    hv1 = jnp.where(m1, av1, r1v)
    m = h0 >= h1
    pk = jnp.where(m, h0, h1)
    pv = jnp.where(m, hv0, hv1)
    qk = jnp.where(m, h1, h0)
    qv = jnp.where(m, hv1, hv0)
    pk, pv = plsc.sort_key_val(pk, pv, descending=True)
    qk, qv = plsc.sort_key_val(qk, qv, descending=True)
    return pk, pv, qk, qv


def _row_topk(buf_ref, r, iotas):
    ks = []
    vs = []
    for j in range(8):
        kj, vj = plsc.sort_key_val(
            buf_ref[r, pl.ds(16 * j, 16)], iotas[j], descending=True)
        ks.append(kj)
        vs.append(vj)
    s = []
    for j in range(4):
        s.append(_merge32(ks[2 * j], vs[2 * j], ks[2 * j + 1], vs[2 * j + 1]))
    t01 = _top32(*s[0], *s[1])
    t23 = _top32(*s[2], *s[3])
    pk, pv, qk, qv = _top32(*t01, *t23)
    mx = jnp.max(pk)
    e0 = jnp.exp(pk - mx)
    e1 = jnp.exp(qk - mx)
    tot = jnp.sum(e0) + jnp.sum(e1)
    return pv, qv, e0 / tot, e1 / tot


def _sc_side(l_hbm, ti_hbm, w_hbm, buf_v, ti_v, w_v, wid, rows_pw, iotas):
    def chunk(c, carry):
        row0 = wid * rows_pw + c * _CH
        pltpu.sync_copy(l_hbm.at[pl.ds(row0, _CH)], buf_v)
        for r in range(_CH):
            pv, qv, w0, w1 = _row_topk(buf_v, r, iotas)
            ti_v[r, pl.ds(0, 16)] = pv
            ti_v[r, pl.ds(16, 16)] = qv
            w_v[r, pl.ds(0, 16)] = w0
            w_v[r, pl.ds(16, 16)] = w1
        pltpu.sync_copy(ti_v, ti_hbm.at[pl.ds(row0, _CH)])
        pltpu.sync_copy(w_v, w_hbm.at[pl.ds(row0, _CH)])
        return carry

    lax.fori_loop(0, rows_pw // _CH, chunk, 0)


def _sc_topk(lA, lB):
    rows = lA.shape[0]
    k = 32
    rows_pw = rows // _NW
    mesh = plsc.VectorSubcoreMesh(core_axis_name="c", subcore_axis_name="s")

    @functools.partial(
        pl.kernel, mesh=mesh,
        out_type=[
            jax.ShapeDtypeStruct((rows, k), jnp.int32),
            jax.ShapeDtypeStruct((rows, k), jnp.float32),
            jax.ShapeDtypeStruct((rows, k), jnp.int32),
            jax.ShapeDtypeStruct((rows, k), jnp.float32),
        ],
        scratch_types=[
            pltpu.VMEM((_CH, 128), jnp.float32),
            pltpu.VMEM((_CH, k), jnp.int32),
            pltpu.VMEM((_CH, k), jnp.float32),
        ],
        compiler_params=pltpu.CompilerParams(needs_layout_passes=False),
    )
    def run(la_hbm, lb_hbm, tia_hbm, wa_hbm, tib_hbm, wb_hbm,
            buf_v, ti_v, w_v):
        wid = lax.axis_index("s") * 2 + lax.axis_index("c")
        base = lax.iota(jnp.int32, 16)
        iotas = [base + 16 * j for j in range(8)]
        _sc_side(la_hbm, tia_hbm, wa_hbm, buf_v, ti_v, w_v, wid, rows_pw, iotas)
        _sc_side(lb_hbm, tib_hbm, wb_hbm, buf_v, ti_v, w_v, wid, rows_pw, iotas)

    return run(lA, lB)


def kernel(X_A, X_B, prototype_base, Wc, bc, WA, bA, WB, bB):
    B, N, D = X_A.shape
    E_dim = prototype_base.shape[0]
    k = max(1, E_dim // 4)
    H = 8
    d_h = D // H
    scale = 1.0 / (H * math.sqrt(d_h))
    nt = min(512, N)

    ctx = pl.pallas_call(
        _ctx_kernel,
        grid=(B,),
        in_specs=[
            pl.BlockSpec((1, N, D), lambda b: (b, 0, 0)),
            pl.BlockSpec((1, N, D), lambda b: (b, 0, 0)),
        ],
        out_specs=pl.BlockSpec((1, 1, 2 * D), lambda b: (b, 0, 0)),
        out_shape=jax.ShapeDtypeStruct((B, 1, 2 * D), jnp.float32),
        interpret=False,
    )(X_A, X_B)
    ctx2 = ctx.reshape(B, 2 * D)

    ec = 2
    Wc3 = Wc.reshape(E_dim, D, 2 * D)
    bc3 = bc.reshape(E_dim, 1, D)
    base3 = prototype_base.reshape(E_dim, 1, D)
    proto = pl.pallas_call(
        _proto_kernel,
        grid=(E_dim // ec,),
        in_specs=[
            pl.BlockSpec((B, 2 * D), lambda e: (0, 0)),
            pl.BlockSpec((ec, D, 2 * D), lambda e: (e, 0, 0)),
            pl.BlockSpec((ec, 1, D), lambda e: (e, 0, 0)),
            pl.BlockSpec((ec, 1, D), lambda e: (e, 0, 0)),
        ],
        out_specs=pl.BlockSpec((ec, B, D), lambda e: (e, 0, 0)),
        out_shape=jax.ShapeDtypeStruct((E_dim, B, D), jnp.float32),
        interpret=False,
    )(ctx2, Wc3, bc3, base3)
    protoT = proto.transpose(1, 2, 0)

    def logits_call(X, W, bias):
        body = functools.partial(_logits_kernel, scale=scale)
        return pl.pallas_call(
            body,
            grid=(B, N // nt),
            in_specs=[
                pl.BlockSpec((1, nt, D), lambda b, n: (b, n, 0)),
                pl.BlockSpec((D, D), lambda b, n: (0, 0)),
                pl.BlockSpec((1, D), lambda b, n: (0, 0)),
                pl.BlockSpec((1, D, E_dim), lambda b, n: (b, 0, 0)),
            ],
            out_specs=pl.BlockSpec((1, nt, E_dim), lambda b, n: (b, n, 0)),
            out_shape=jax.ShapeDtypeStruct((B, N, E_dim), jnp.float32),
            interpret=False,
        )(X, W, bias, protoT)

    lA = logits_call(X_A, WA, bA.reshape(1, D)).reshape(B * N, E_dim)
    lB = logits_call(X_B, WB, bB.reshape(1, D)).reshape(B * N, E_dim)
    tiA, wA, tiB, wB = _sc_topk(lA, lB)
    return (tiA.reshape(B, N, k), wA.reshape(B, N, k),
            tiB.reshape(B, N, k), wB.reshape(B, N, k), E_dim)


# speedup vs baseline: 1.5580x; 1.0927x over previous
import functools
import math

import jax
import jax.numpy as jnp
from jax import lax
from jax.experimental import pallas as pl
from jax.experimental.pallas import tpu as pltpu
from jax.experimental.pallas import tpu_sc as plsc

_NW = 32
_CH = 16


def _ctx_kernel(xa_ref, xb_ref, out_ref):
    d = xa_ref.shape[2]
    out_ref[0, 0, :d] = jnp.mean(xa_ref[0], axis=0)
    out_ref[0, 0, d:] = jnp.mean(xb_ref[0], axis=0)


def _proto_kernel(ctx_ref, wc_ref, bc_ref, base_ref, out_ref):
    for i in range(wc_ref.shape[0]):
        off = jax.lax.dot_general(
            ctx_ref[...], wc_ref[i], (((1,), (1,)), ((), ())),
            preferred_element_type=jnp.float32)
        out_ref[i] = off + bc_ref[i] + base_ref[i]


def _logits_kernel(x_ref, w_ref, b_ref, p_ref, out_ref, *, scale):
    x = x_ref[0]
    xp = jax.lax.dot_general(
        x, w_ref[...], (((1,), (1,)), ((), ())),
        preferred_element_type=jnp.float32) + b_ref[...]
    out_ref[0] = jnp.dot(xp, p_ref[0], preferred_element_type=jnp.float32) * scale


def _merge32(ka, va, kb, vb):
    rk = jnp.flip(kb, 0)
    rv = jnp.flip(vb, 0)
    m = ka >= rk
    hk = jnp.where(m, ka, rk)
    hv = jnp.where(m, va, rv)
    lk = jnp.where(m, rk, ka)
    lv = jnp.where(m, rv, va)
    hk, hv = plsc.sort_key_val(hk, hv, descending=True)
    lk, lv = plsc.sort_key_val(lk, lv, descending=True)
    return hk, hv, lk, lv


def _top32(a0, av0, a1, av1, b0, bv0, b1, bv1):
    r0k = jnp.flip(b1, 0)
    r0v = jnp.flip(bv1, 0)
    r1k = jnp.flip(b0, 0)
    r1v = jnp.flip(bv0, 0)
    m0 = a0 >= r0k
    h0 = jnp.where(m0, a0, r0k)
    hv0 = jnp.where(m0, av0, r0v)
    m1 = a1 >= r1k
    h1 = jnp.where(m1, a1, r1k)
    hv1 = jnp.where(m1, av1, r1v)
    m = h0 >= h1
    pk = jnp.where(m, h0, h1)
    pv = jnp.where(m, hv0, hv1)
    qk = jnp.where(m, h1, h0)
    qv = jnp.where(m, hv1, hv0)
    pk, pv = plsc.sort_key_val(pk, pv, descending=True)
    qk, qv = plsc.sort_key_val(qk, qv, descending=True)
    return pk, pv, qk, qv


def _row_topk(buf_ref, r, iotas):
    ks = []
    vs = []
    for j in range(8):
        kj, vj = plsc.sort_key_val(
            buf_ref[r, pl.ds(16 * j, 16)], iotas[j], descending=True)
        ks.append(kj)
        vs.append(vj)
    s = []
    for j in range(4):
        s.append(_merge32(ks[2 * j], vs[2 * j], ks[2 * j + 1], vs[2 * j + 1]))
    t01 = _top32(*s[0], *s[1])
    t23 = _top32(*s[2], *s[3])
    pk, pv, qk, qv = _top32(*t01, *t23)
    mx = jnp.max(pk)
    e0 = jnp.exp(pk - mx)
    e1 = jnp.exp(qk - mx)
    tot = jnp.sum(e0) + jnp.sum(e1)
    return pv, qv, e0 / tot, e1 / tot


def _sc_side(l_hbm, ti_hbm, w_hbm, buf_v, ti_v, w_v, wid, rows_pw, iotas):
    def chunk(c, carry):
        row0 = wid * rows_pw + c * _CH
        pltpu.sync_copy(l_hbm.at[pl.ds(row0, _CH)], buf_v)
        for r in range(_CH):
            pv, qv, w0, w1 = _row_topk(buf_v, r, iotas)
            ti_v[r, pl.ds(0, 16)] = pv
            ti_v[r, pl.ds(16, 16)] = qv
            w_v[r, pl.ds(0, 16)] = w0
            w_v[r, pl.ds(16, 16)] = w1
        pltpu.sync_copy(ti_v, ti_hbm.at[pl.ds(row0, _CH)])
        pltpu.sync_copy(w_v, w_hbm.at[pl.ds(row0, _CH)])
        return carry

    lax.fori_loop(0, rows_pw // _CH, chunk, 0)


def _sc_topk(l):
    rows = l.shape[0]
    k = 32
    rows_pw = rows // _NW
    mesh = plsc.VectorSubcoreMesh(core_axis_name="c", subcore_axis_name="s")

    @functools.partial(
        pl.kernel, mesh=mesh,
        out_type=[
            jax.ShapeDtypeStruct((rows, k), jnp.int32),
            jax.ShapeDtypeStruct((rows, k), jnp.float32),
        ],
        scratch_types=[
            pltpu.VMEM((_CH, 128), jnp.float32),
            pltpu.VMEM((_CH, k), jnp.int32),
            pltpu.VMEM((_CH, k), jnp.float32),
        ],
        compiler_params=pltpu.CompilerParams(needs_layout_passes=False),
    )
    def run(l_hbm, ti_hbm, w_hbm, buf_v, ti_v, w_v):
        wid = lax.axis_index("s") * 2 + lax.axis_index("c")
        base = lax.iota(jnp.int32, 16)
        iotas = [base + 16 * j for j in range(8)]
        _sc_side(l_hbm, ti_hbm, w_hbm, buf_v, ti_v, w_v, wid, rows_pw, iotas)

    return run(l)


def kernel(X_A, X_B, prototype_base, Wc, bc, WA, bA, WB, bB):
    B, N, D = X_A.shape
    E_dim = prototype_base.shape[0]
    k = max(1, E_dim // 4)
    H = 8
    d_h = D // H
    scale = 1.0 / (H * math.sqrt(d_h))
    nt = min(512, N)

    ctx = pl.pallas_call(
        _ctx_kernel,
        grid=(B,),
        in_specs=[
            pl.BlockSpec((1, N, D), lambda b: (b, 0, 0)),
            pl.BlockSpec((1, N, D), lambda b: (b, 0, 0)),
        ],
        out_specs=pl.BlockSpec((1, 1, 2 * D), lambda b: (b, 0, 0)),
        out_shape=jax.ShapeDtypeStruct((B, 1, 2 * D), jnp.float32),
        interpret=False,
    )(X_A, X_B)
    ctx2 = ctx.reshape(B, 2 * D)

    ec = 2
    Wc3 = Wc.reshape(E_dim, D, 2 * D)
    bc3 = bc.reshape(E_dim, 1, D)
    base3 = prototype_base.reshape(E_dim, 1, D)
    proto = pl.pallas_call(
        _proto_kernel,
        grid=(E_dim // ec,),
        in_specs=[
            pl.BlockSpec((B, 2 * D), lambda e: (0, 0)),
            pl.BlockSpec((ec, D, 2 * D), lambda e: (e, 0, 0)),
            pl.BlockSpec((ec, 1, D), lambda e: (e, 0, 0)),
            pl.BlockSpec((ec, 1, D), lambda e: (e, 0, 0)),
        ],
        out_specs=pl.BlockSpec((ec, B, D), lambda e: (e, 0, 0)),
        out_shape=jax.ShapeDtypeStruct((E_dim, B, D), jnp.float32),
        interpret=False,
    )(ctx2, Wc3, bc3, base3)
    protoT = proto.transpose(1, 2, 0)

    def logits_call(X, W, bias):
        body = functools.partial(_logits_kernel, scale=scale)
        return pl.pallas_call(
            body,
            grid=(B, N // nt),
            in_specs=[
                pl.BlockSpec((1, nt, D), lambda b, n: (b, n, 0)),
                pl.BlockSpec((D, D), lambda b, n: (0, 0)),
                pl.BlockSpec((1, D), lambda b, n: (0, 0)),
                pl.BlockSpec((1, D, E_dim), lambda b, n: (b, 0, 0)),
            ],
            out_specs=pl.BlockSpec((1, nt, E_dim), lambda b, n: (b, n, 0)),
            out_shape=jax.ShapeDtypeStruct((B, N, E_dim), jnp.float32),
            interpret=False,
        )(X, W, bias, protoT)

    lA = logits_call(X_A, WA, bA.reshape(1, D)).reshape(B * N, E_dim)
    tiA, wA = _sc_topk(lA)
    lB = logits_call(X_B, WB, bB.reshape(1, D)).reshape(B * N, E_dim)
    tiB, wB = _sc_topk(lB)
    return (tiA.reshape(B, N, k), wA.reshape(B, N, k),
            tiB.reshape(B, N, k), wB.reshape(B, N, k), E_dim)


# speedup vs baseline: 1.5903x; 1.0207x over previous
import functools
import math

import jax
import jax.numpy as jnp
from jax import lax
from jax.experimental import pallas as pl
from jax.experimental.pallas import tpu as pltpu
from jax.experimental.pallas import tpu_sc as plsc

_NW = 32
_CH = 16


def _ctx_kernel(xa_ref, xb_ref, out_ref):
    d = xa_ref.shape[2]
    out_ref[0, 0, :d] = jnp.mean(xa_ref[0], axis=0)
    out_ref[0, 0, d:] = jnp.mean(xb_ref[0], axis=0)


def _proto_kernel(ctx_ref, wc_ref, bc_ref, base_ref, out_ref):
    for i in range(wc_ref.shape[0]):
        off = jax.lax.dot_general(
            ctx_ref[...], wc_ref[i], (((1,), (1,)), ((), ())),
            preferred_element_type=jnp.float32)
        out_ref[i] = off + bc_ref[i] + base_ref[i]


def _logits_kernel(x_ref, w_ref, b_ref, p_ref, out_ref, *, scale):
    x = x_ref[0]
    xp = jax.lax.dot_general(
        x, w_ref[...], (((1,), (1,)), ((), ())),
        preferred_element_type=jnp.float32) + b_ref[...]
    out_ref[0] = jnp.dot(xp, p_ref[0], preferred_element_type=jnp.float32) * scale


def _merge32(ka, va, kb, vb):
    rk = jnp.flip(kb, 0)
    rv = jnp.flip(vb, 0)
    m = ka >= rk
    hk = jnp.where(m, ka, rk)
    hv = jnp.where(m, va, rv)
    lk = jnp.where(m, rk, ka)
    lv = jnp.where(m, rv, va)
    hk, hv = plsc.sort_key_val(hk, hv, descending=True)
    lk, lv = plsc.sort_key_val(lk, lv, descending=True)
    return hk, hv, lk, lv


def _top32(a0, av0, a1, av1, b0, bv0, b1, bv1):
    r0k = jnp.flip(b1, 0)
    r0v = jnp.flip(bv1, 0)
    r1k = jnp.flip(b0, 0)
    r1v = jnp.flip(bv0, 0)
    m0 = a0 >= r0k
    h0 = jnp.where(m0, a0, r0k)
    hv0 = jnp.where(m0, av0, r0v)
    m1 = a1 >= r1k
    h1 = jnp.where(m1, a1, r1k)
    hv1 = jnp.where(m1, av1, r1v)
    m = h0 >= h1
    pk = jnp.where(m, h0, h1)
    pv = jnp.where(m, hv0, hv1)
    qk = jnp.where(m, h1, h0)
    qv = jnp.where(m, hv1, hv0)
    pk, pv = plsc.sort_key_val(pk, pv, descending=True)
    qk, qv = plsc.sort_key_val(qk, qv, descending=True)
    return pk, pv, qk, qv


def _row_topk(buf_ref, r, iotas):
    ks = []
    vs = []
    for j in range(8):
        kj, vj = plsc.sort_key_val(
            buf_ref[r, pl.ds(16 * j, 16)], iotas[j], descending=True)
        ks.append(kj)
        vs.append(vj)
    s = []
    for j in range(4):
        s.append(_merge32(ks[2 * j], vs[2 * j], ks[2 * j + 1], vs[2 * j + 1]))
    t01 = _top32(*s[0], *s[1])
    t23 = _top32(*s[2], *s[3])
    pk, pv, qk, qv = _top32(*t01, *t23)
    mx = jnp.max(pk)
    e0 = jnp.exp(pk - mx)
    e1 = jnp.exp(qk - mx)
    tot = jnp.sum(e0) + jnp.sum(e1)
    return pv, qv, e0 / tot, e1 / tot


def _sc_side(l_hbm, ti_hbm, w_hbm, buf_v, ti_v, w_v, wid, rows_pw, iotas):
    def chunk(c, carry):
        row0 = wid * rows_pw + c * _CH
        pltpu.sync_copy(l_hbm.at[pl.ds(row0, _CH)], buf_v)
        for r in range(_CH):
            pv, qv, w0, w1 = _row_topk(buf_v, r, iotas)
            ti_v[r, pl.ds(0, 16)] = pv
            ti_v[r, pl.ds(16, 16)] = qv
            w_v[r, pl.ds(0, 16)] = w0
            w_v[r, pl.ds(16, 16)] = w1
        pltpu.sync_copy(ti_v, ti_hbm.at[pl.ds(row0, _CH)])
        pltpu.sync_copy(w_v, w_hbm.at[pl.ds(row0, _CH)])
        return carry

    lax.fori_loop(0, rows_pw // _CH, chunk, 0)


def _sc_topk(l):
    rows = l.shape[0]
    k = 32
    rows_pw = rows // _NW
    mesh = plsc.VectorSubcoreMesh(core_axis_name="c", subcore_axis_name="s")

    @functools.partial(
        pl.kernel, mesh=mesh,
        out_type=[
            jax.ShapeDtypeStruct((rows, k), jnp.int32),
            jax.ShapeDtypeStruct((rows, k), jnp.float32),
        ],
        scratch_types=[
            pltpu.VMEM((_CH, 128), jnp.float32),
            pltpu.VMEM((_CH, k), jnp.int32),
            pltpu.VMEM((_CH, k), jnp.float32),
        ],
        compiler_params=pltpu.CompilerParams(needs_layout_passes=False),
    )
    def run(l_hbm, ti_hbm, w_hbm, buf_v, ti_v, w_v):
        wid = lax.axis_index("s") * 2 + lax.axis_index("c")
        base = lax.iota(jnp.int32, 16)
        iotas = [base + 16 * j for j in range(8)]
        _sc_side(l_hbm, ti_hbm, w_hbm, buf_v, ti_v, w_v, wid, rows_pw, iotas)

    return run(l)


def kernel(X_A, X_B, prototype_base, Wc, bc, WA, bA, WB, bB):
    B, N, D = X_A.shape
    E_dim = prototype_base.shape[0]
    k = max(1, E_dim // 4)
    H = 8
    d_h = D // H
    scale = 1.0 / (H * math.sqrt(d_h))
    nt = min(1024, N)

    ctx = pl.pallas_call(
        _ctx_kernel,
        grid=(B,),
        in_specs=[
            pl.BlockSpec((1, N, D), lambda b: (b, 0, 0)),
            pl.BlockSpec((1, N, D), lambda b: (b, 0, 0)),
        ],
        out_specs=pl.BlockSpec((1, 1, 2 * D), lambda b: (b, 0, 0)),
        out_shape=jax.ShapeDtypeStruct((B, 1, 2 * D), jnp.float32),
        interpret=False,
    )(X_A, X_B)
    ctx2 = ctx.reshape(B, 2 * D)

    ec = 2
    Wc3 = Wc.reshape(E_dim, D, 2 * D)
    bc3 = bc.reshape(E_dim, 1, D)
    base3 = prototype_base.reshape(E_dim, 1, D)
    proto = pl.pallas_call(
        _proto_kernel,
        grid=(E_dim // ec,),
        in_specs=[
            pl.BlockSpec((B, 2 * D), lambda e: (0, 0)),
            pl.BlockSpec((ec, D, 2 * D), lambda e: (e, 0, 0)),
            pl.BlockSpec((ec, 1, D), lambda e: (e, 0, 0)),
            pl.BlockSpec((ec, 1, D), lambda e: (e, 0, 0)),
        ],
        out_specs=pl.BlockSpec((ec, B, D), lambda e: (e, 0, 0)),
        out_shape=jax.ShapeDtypeStruct((E_dim, B, D), jnp.float32),
        interpret=False,
    )(ctx2, Wc3, bc3, base3)
    protoT = proto.transpose(1, 2, 0)

    def logits_call(X, W, bias):
        body = functools.partial(_logits_kernel, scale=scale)
        return pl.pallas_call(
            body,
            grid=(B, N // nt),
            in_specs=[
                pl.BlockSpec((1, nt, D), lambda b, n: (b, n, 0)),
                pl.BlockSpec((D, D), lambda b, n: (0, 0)),
                pl.BlockSpec((1, D), lambda b, n: (0, 0)),
                pl.BlockSpec((1, D, E_dim), lambda b, n: (b, 0, 0)),
            ],
            out_specs=pl.BlockSpec((1, nt, E_dim), lambda b, n: (b, n, 0)),
            out_shape=jax.ShapeDtypeStruct((B, N, E_dim), jnp.float32),
            interpret=False,
        )(X, W, bias, protoT)

    lA = logits_call(X_A, WA, bA.reshape(1, D)).reshape(B * N, E_dim)
    tiA, wA = _sc_topk(lA)
    lB = logits_call(X_B, WB, bB.reshape(1, D)).reshape(B * N, E_dim)
    tiB, wB = _sc_topk(lB)
    return (tiA.reshape(B, N, k), wA.reshape(B, N, k),
            tiB.reshape(B, N, k), wB.reshape(B, N, k), E_dim)
